# baseline (speedup 1.0000x reference)
import functools

import jax
import jax.numpy as jnp
from jax import lax
from jax.experimental import pallas as pl
from jax.experimental.pallas import tpu as pltpu
from jax.experimental.pallas import tpu_sc as plsc

_C = 400
_NW = 32


def _mlp_body(x_ref, w1_ref, b1_ref, w2_ref, b2_ref, w3_ref, b3_ref, out_ref):
    h = jnp.dot(x_ref[...], w1_ref[...], preferred_element_type=jnp.float32)
    h = jnp.maximum(h + b1_ref[...], 0.0)
    d = jnp.dot(h, w2_ref[...], preferred_element_type=jnp.float32) + b2_ref[...]
    out_ref[...] = (
        jnp.dot(d, w3_ref[...], preferred_element_type=jnp.float32) + b3_ref[...]
    )


def _mlp(x, W1, b1, W2, b2, W3, b3):
    n, d = x.shape
    h = W1.shape[1]
    h2 = W2.shape[1]
    out_d = W3.shape[1]
    blk = 1000
    return pl.pallas_call(
        _mlp_body,
        grid=(n // blk,),
        in_specs=[
            pl.BlockSpec((blk, d), lambda i: (i, 0)),
            pl.BlockSpec((d, h), lambda i: (0, 0)),
            pl.BlockSpec((1, h), lambda i: (0, 0)),
            pl.BlockSpec((h, h2), lambda i: (0, 0)),
            pl.BlockSpec((1, h2), lambda i: (0, 0)),
            pl.BlockSpec((h2, out_d), lambda i: (0, 0)),
            pl.BlockSpec((1, out_d), lambda i: (0, 0)),
        ],
        out_specs=pl.BlockSpec((blk, out_d), lambda i: (i, 0)),
        out_shape=jax.ShapeDtypeStruct((n, out_d), jnp.float32),
    )(x, W1, b1.reshape(1, -1), W2, b2.reshape(1, -1), W3, b3.reshape(1, -1))


_NB = 4


def _sc_gather(x_dnn, x, idx_l, idx_r, n_chunks, c, ipw, slot):
    n_pad = idx_l.shape[0]
    d = x.shape[1]
    mesh = plsc.VectorSubcoreMesh(core_axis_name="c", subcore_axis_name="s")

    @functools.partial(
        pl.kernel,
        mesh=mesh,
        out_type=[
            jax.ShapeDtypeStruct((2, n_chunks, c, d), jnp.float32),
            jax.ShapeDtypeStruct((2, n_chunks, c, d), jnp.float32),
        ],
        scratch_types=[
            pltpu.VMEM((c,), jnp.int32),
            pltpu.VMEM((c, d), jnp.float32),
            pltpu.VMEM((c, d), jnp.float32),
            pltpu.SemaphoreType.DMA,
            pltpu.SemaphoreType.DMA,
            pltpu.SemaphoreType.DMA,
        ],
    )
    def k(dnn_hbm, x_hbm, idxl_hbm, idxr_hbm, emb_hbm, feat_hbm,
          idx_v, rows0, rows1, gsem, wsem0, wsem1):
        wid = lax.axis_index("s") * 2 + lax.axis_index("c")
        bufs = (rows0, rows1)
        wsems = (wsem0, wsem1)

        jobs = (
            (dnn_hbm, idxl_hbm, emb_hbm, 0),
            (dnn_hbm, idxr_hbm, emb_hbm, 1),
            (x_hbm, idxl_hbm, feat_hbm, 0),
            (x_hbm, idxr_hbm, feat_hbm, 1),
        )
        T = 4 * ipw

        def parts(t):
            j, i = divmod(t, ipw)
            table, idx_hbm, out_hbm, side = jobs[j]
            ch = wid + i * _NW
            live = i * _NW + _NW - 1 < n_chunks
            return table, idx_hbm.at[ch], out_hbm, side, ch, live

        def w_copy(t):
            _, _, out_hbm, side, ch, _ = parts(t)
            b = t % 2
            return pltpu.make_async_copy(bufs[b], out_hbm.at[side, ch],
                                         wsems[b])

        def maybe(t, fn):
            _, _, _, _, ch, live = parts(t)
            if live:
                fn()
            else:
                pl.when(ch < n_chunks)(fn)

        for t in range(T):
            table, idx_src, out_hbm, side, _, _ = parts(t)
            b = t % 2
            pltpu.sync_copy(idx_src, idx_v)
            pltpu.async_copy(table.at[idx_v], bufs[b], gsem).wait()
            pltpu.sync_copy(bufs[b], out_hbm.at[side, wid])

    return k(x_dnn, x, idx_l, idx_r)


def kernel(x, edge_index, pair_idxs_left, pair_idxs_right, y, W_lin, b_lin,
           W_gat, a_src, a_dst, b_gat, W1, b1, W2, b2, W3, b3):
    p = pair_idxs_left.shape[0]
    x_dnn = _mlp(x, W1, b1, W2, b2, W3, b3)
    n_chunks = p // _C
    n_pad = ((n_chunks + _NW - 1) // _NW) * _NW
    ipw = n_pad // _NW
    slot = ((_C + 127) // 128) * 128

    def prep(idx):
        return jnp.pad(idx.reshape(-1, _C), ((0, n_pad - n_chunks), (0, 0)))

    emb, feat = _sc_gather(x_dnn, x, prep(pair_idxs_left),
                           prep(pair_idxs_right), n_chunks, _C, ipw, slot)
    return (emb.reshape(2, p, -1), feat.reshape(2, p, -1), y)


# speedup vs baseline: 3.5428x; 3.5428x over previous
import functools

import jax
import jax.numpy as jnp
from jax import lax
from jax.experimental import pallas as pl
from jax.experimental.pallas import tpu as pltpu
from jax.experimental.pallas import tpu_sc as plsc

_C = 400
_NW = 32


def _mlp_body(x_ref, w1_ref, b1_ref, w2_ref, b2_ref, w3_ref, b3_ref, out_ref):
    h = jnp.dot(x_ref[...], w1_ref[...], preferred_element_type=jnp.float32)
    h = jnp.maximum(h + b1_ref[...], 0.0)
    d = jnp.dot(h, w2_ref[...], preferred_element_type=jnp.float32) + b2_ref[...]
    out_ref[...] = (
        jnp.dot(d, w3_ref[...], preferred_element_type=jnp.float32) + b3_ref[...]
    )


def _mlp(x, W1, b1, W2, b2, W3, b3):
    n, d = x.shape
    h = W1.shape[1]
    h2 = W2.shape[1]
    out_d = W3.shape[1]
    blk = 1000
    return pl.pallas_call(
        _mlp_body,
        grid=(n // blk,),
        in_specs=[
            pl.BlockSpec((blk, d), lambda i: (i, 0)),
            pl.BlockSpec((d, h), lambda i: (0, 0)),
            pl.BlockSpec((1, h), lambda i: (0, 0)),
            pl.BlockSpec((h, h2), lambda i: (0, 0)),
            pl.BlockSpec((1, h2), lambda i: (0, 0)),
            pl.BlockSpec((h2, out_d), lambda i: (0, 0)),
            pl.BlockSpec((1, out_d), lambda i: (0, 0)),
        ],
        out_specs=pl.BlockSpec((blk, out_d), lambda i: (i, 0)),
        out_shape=jax.ShapeDtypeStruct((n, out_d), jnp.float32),
    )(x, W1, b1.reshape(1, -1), W2, b2.reshape(1, -1), W3, b3.reshape(1, -1))


_NB = 4


def _sc_gather(x_dnn, x, idx_l, idx_r, n_chunks, c, ipw, slot):
    del slot
    n_pad = idx_l.shape[0]
    d = x.shape[1]
    mesh = plsc.VectorSubcoreMesh(core_axis_name="c", subcore_axis_name="s")

    @functools.partial(
        pl.kernel,
        mesh=mesh,
        out_type=[
            jax.ShapeDtypeStruct((2, n_chunks, c, d), jnp.float32),
            jax.ShapeDtypeStruct((2, n_chunks, c, d), jnp.float32),
        ],
        scratch_types=[
            pltpu.VMEM((c,), jnp.int32),
            pltpu.VMEM((c, d), jnp.float32),
            pltpu.VMEM((c, d), jnp.float32),
            pltpu.SemaphoreType.DMA,
            pltpu.SemaphoreType.DMA,
            pltpu.SemaphoreType.DMA,
        ],
    )
    def k(dnn_hbm, x_hbm, idxl_hbm, idxr_hbm, emb_hbm, feat_hbm,
          idx_v, rows0, rows1, gsem, wsem0, wsem1):
        wid = lax.axis_index("s") * 2 + lax.axis_index("c")
        bufs = (rows0, rows1)
        wsems = (wsem0, wsem1)

        jobs = (
            (dnn_hbm, idxl_hbm, emb_hbm, 0),
            (dnn_hbm, idxr_hbm, emb_hbm, 1),
            (x_hbm, idxl_hbm, feat_hbm, 0),
            (x_hbm, idxr_hbm, feat_hbm, 1),
        )
        T = 4 * ipw

        def parts(t):
            j, i = divmod(t, ipw)
            table, idx_hbm, out_hbm, side = jobs[j]
            ch = wid + i * _NW
            live = i * _NW + _NW - 1 < n_chunks
            return table, idx_hbm.at[ch], out_hbm, side, ch, live

        def w_copy(t):
            _, _, out_hbm, side, ch, _ = parts(t)
            b = t % 2
            return pltpu.make_async_copy(bufs[b], out_hbm.at[side, ch],
                                         wsems[b])

        def maybe(t, fn):
            _, _, _, _, ch, live = parts(t)
            if live:
                fn()
            else:
                pl.when(ch < n_chunks)(fn)

        for t in range(T):
            table, idx_src, _, _, _, _ = parts(t)
            b = t % 2
            if t >= 2:
                maybe(t - 2, w_copy(t - 2).wait)

            def step(table=table, idx_src=idx_src, b=b, t=t):
                pltpu.sync_copy(idx_src, idx_v)
                pltpu.async_copy(table.at[idx_v], bufs[b], gsem).wait()
                w_copy(t).start()

            maybe(t, step)
        for t in range(T - 2, T):
            maybe(t, w_copy(t).wait)

    return k(x_dnn, x, idx_l, idx_r)


def kernel(x, edge_index, pair_idxs_left, pair_idxs_right, y, W_lin, b_lin,
           W_gat, a_src, a_dst, b_gat, W1, b1, W2, b2, W3, b3):
    p = pair_idxs_left.shape[0]
    x_dnn = _mlp(x, W1, b1, W2, b2, W3, b3)
    n_chunks = p // _C
    n_pad = ((n_chunks + _NW - 1) // _NW) * _NW
    ipw = n_pad // _NW
    slot = ((_C + 127) // 128) * 128

    def prep(idx):
        return jnp.pad(idx.reshape(-1, _C), ((0, n_pad - n_chunks), (0, 0)))

    emb, feat = _sc_gather(x_dnn, x, prep(pair_idxs_left),
                           prep(pair_idxs_right), n_chunks, _C, ipw, slot)
    return (emb.reshape(2, p, -1), feat.reshape(2, p, -1), y)
